# baseline (speedup 1.0000x reference)
import jax
import jax.numpy as jnp
from jax.experimental import pallas as pl


def _body(edges_ref, atoms_ref, bonds_ref, wa_ref, wb_ref, bias_ref, out_ref):
    edges = edges_ref[...]
    mask = jnp.any(edges != -1, axis=-1, keepdims=True)
    maskf = mask.astype(jnp.float32)
    pa = jnp.sum(atoms_ref[...] * maskf, axis=1)
    pb = jnp.sum(bonds_ref[...] * maskf, axis=1)
    cnt = jnp.sum(maskf[:, :, 0], axis=1)
    out = jnp.dot(pa, wa_ref[...], preferred_element_type=jnp.float32)
    out += jnp.dot(pb, wb_ref[...], preferred_element_type=jnp.float32)
    out += cnt[:, None] * bias_ref[...]
    out_ref[...] = out


def kernel(atoms, bonds, edges, W, b):
    B, A, AF = atoms.shape
    DEG, BF = bonds.shape[2], bonds.shape[3]
    FP = W.shape[1]
    BB = 32

    bonds2 = bonds.reshape(B, A, DEG * BF)
    wa = W[:AF]
    wb = jnp.tile(W[AF:], (DEG, 1))
    bias = b.reshape(1, FP)

    return pl.pallas_call(
        _body,
        grid=(B // BB,),
        in_specs=[
            pl.BlockSpec((BB, A, DEG), lambda i: (i, 0, 0)),
            pl.BlockSpec((BB, A, AF), lambda i: (i, 0, 0)),
            pl.BlockSpec((BB, A, DEG * BF), lambda i: (i, 0, 0)),
            pl.BlockSpec((AF, FP), lambda i: (0, 0)),
            pl.BlockSpec((DEG * BF, FP), lambda i: (0, 0)),
            pl.BlockSpec((1, FP), lambda i: (0, 0)),
        ],
        out_specs=pl.BlockSpec((BB, FP), lambda i: (i, 0)),
        out_shape=jax.ShapeDtypeStruct((B, FP), jnp.float32),
    )(edges, atoms, bonds2, wa, wb, bias)


# speedup vs baseline: 3.9208x; 3.9208x over previous
import jax
import jax.numpy as jnp
from jax.experimental import pallas as pl


def _body(edges_ref, atoms_ref, bonds_ref, wa_ref, wb_ref, bias_ref, out_ref):
    edges = edges_ref[...]
    mask = jnp.any(edges != -1, axis=1)
    maskf = mask.astype(jnp.float32)
    pb = jnp.sum(bonds_ref[...] * maskf[:, None, None, :], axis=(0, 1))
    maskt = maskf.T
    pa = jnp.sum(atoms_ref[...] * maskt[:, :, None], axis=1)
    cnt = jnp.sum(maskt, axis=1)
    out = jnp.dot(pa, wa_ref[...], preferred_element_type=jnp.float32)
    out += jax.lax.dot_general(pb, wb_ref[...], (((0,), (0,)), ((), ())),
                               preferred_element_type=jnp.float32)
    out += cnt[:, None] * bias_ref[...]
    out_ref[...] = out


def kernel(atoms, bonds, edges, W, b):
    B, A, AF = atoms.shape
    DEG, BF = bonds.shape[2], bonds.shape[3]
    FP = W.shape[1]
    BB = 128

    bonds_t = jnp.transpose(bonds, (1, 2, 3, 0))
    edges_t = jnp.transpose(edges, (1, 2, 0))
    wa = W[:AF]
    wb = W[AF:]
    bias = b.reshape(1, FP)

    return pl.pallas_call(
        _body,
        grid=(B // BB,),
        in_specs=[
            pl.BlockSpec((A, DEG, BB), lambda i: (0, 0, i)),
            pl.BlockSpec((BB, A, AF), lambda i: (i, 0, 0)),
            pl.BlockSpec((A, DEG, BF, BB), lambda i: (0, 0, 0, i)),
            pl.BlockSpec((AF, FP), lambda i: (0, 0)),
            pl.BlockSpec((BF, FP), lambda i: (0, 0)),
            pl.BlockSpec((1, FP), lambda i: (0, 0)),
        ],
        out_specs=pl.BlockSpec((BB, FP), lambda i: (i, 0)),
        out_shape=jax.ShapeDtypeStruct((B, FP), jnp.float32),
    )(edges_t, atoms, bonds_t, wa, wb, bias)
